# baseline (speedup 1.0000x reference)
import functools

import jax
import jax.numpy as jnp
from jax import lax
from jax.experimental import pallas as pl
from jax.experimental.pallas import tpu as pltpu
from jax.experimental.pallas import tpu_sc as plsc

NBUF = 2
LANE = 16
RUNROLL = 8


@functools.partial(
    jax.jit, static_argnames=("batch", "hist", "D", "vocab", "num_cores", "num_subcores")
)
def _sc_embedding_gather(idx_flat, table, *, batch, hist, D, vocab, num_cores, num_subcores):
    mesh = plsc.VectorSubcoreMesh(core_axis_name="c", subcore_axis_name="s")
    num_workers = num_cores * num_subcores
    rows_per_w = batch // num_workers
    idx_per_w = rows_per_w * hist
    n_vec = D // LANE
    n_rgrp = hist // RUNROLL
    assert hist % RUNROLL == 0 and D % LANE == 0

    @functools.partial(
        pl.kernel,
        mesh=mesh,
        out_type=(),
        scratch_types=[
            pltpu.VMEM((idx_per_w + LANE,), jnp.int32),
            pltpu.VMEM((vocab, D), jnp.float32),
            pltpu.VMEM((NBUF, hist, D), jnp.float32),
            pltpu.SemaphoreType.DMA((NBUF,)),
        ],
    )
    def k(idx_hbm, table_hbm, out_hbm, idx_v, table_v, rows_v, wsem):
        wid = lax.axis_index("s") * num_cores + lax.axis_index("c")
        row_base = wid * rows_per_w
        pltpu.sync_copy(table_hbm, table_v)
        pltpu.sync_copy(
            idx_hbm.at[pl.ds(row_base * hist, idx_per_w)],
            idx_v.at[pl.ds(0, idx_per_w)],
        )

        def fill_rows(i, b, r0, nrows):
            idx16 = idx_v[pl.ds(i * hist + r0, LANE)]
            for u in range(nrows):
                t = idx16[u]
                for c in range(n_vec):
                    rows_v[b, r0 + u, pl.ds(c * LANE, LANE)] = table_v[
                        t, pl.ds(c * LANE, LANE)
                    ]

        def fill_row(i, b):
            def rgrp(g, carry):
                fill_rows(i, b, g * LANE, LANE)
                return carry

            lax.fori_loop(0, hist // LANE, rgrp, 0)
            tail0 = (hist // LANE) * LANE
            if hist - tail0:
                fill_rows(i, b, tail0, hist - tail0)

        def start_write(i, b):
            pltpu.async_copy(rows_v.at[b], out_hbm.at[row_base + i], wsem.at[b])

        def wait_write(b):
            pltpu.make_async_copy(rows_v.at[b], out_hbm.at[0], wsem.at[b]).wait()

        for b in range(NBUF):
            fill_row(b, b)
            start_write(b, b)

        def body(i, carry):
            b = lax.rem(i, NBUF)
            wait_write(b)
            fill_row(i, b)
            start_write(i, b)
            return carry

        lax.fori_loop(NBUF, rows_per_w, body, 0)

        for b in range(NBUF):
            wait_write(b)

    out_ref = jax.new_ref(jnp.empty((batch, hist, D), jnp.float32))
    k(idx_flat, table, out_ref)
    return jax.freeze(out_ref)


def kernel(location, table):
    batch, hist = location.shape
    vocab, D = table.shape
    info = plsc.get_sparse_core_info()
    assert batch % (info.num_cores * info.num_subcores) == 0
    return _sc_embedding_gather(
        location.astype(jnp.int32).reshape(-1),
        table.astype(jnp.float32),
        batch=batch,
        hist=hist,
        D=D,
        vocab=vocab,
        num_cores=info.num_cores,
        num_subcores=info.num_subcores,
    )
